# baseline (speedup 1.0000x reference)
import jax
import jax.numpy as jnp
from jax.experimental import pallas as pl
from jax.experimental.pallas import tpu as pltpu

N = 5000
NP = 5120
NB = 40
RS = 64
MLOG = 13
C = 128
SIGMA = 0.5
SCORE_THRESH = 0.05
K = 300


def _flat_iota():
    return (jax.lax.broadcasted_iota(jnp.int32, (RS, C), 0) * C
            + jax.lax.broadcasted_iota(jnp.int32, (RS, C), 1))


def _partner(a, j):
    if j < C:
        lo = jnp.roll(a, -j, axis=1)
        hi = jnp.roll(a, j, axis=1)
        lane = jax.lax.broadcasted_iota(jnp.int32, (RS, C), 1)
        bit = (lane & j) == 0
    else:
        r = j // C
        lo = jnp.roll(a, -r, axis=0)
        hi = jnp.roll(a, r, axis=0)
        row = jax.lax.broadcasted_iota(jnp.int32, (RS, C), 0)
        bit = (row & r) == 0
    return jnp.where(bit, lo, hi)


def _bitonic(arrays, before):
    flat = _flat_iota()
    for km in range(1, MLOG + 1):
        k = 1 << km
        dirm = (flat & k) != 0
        for jm in range(km - 1, -1, -1):
            j = 1 << jm
            flip = ((flat & j) != 0) != dirm
            ps = [_partner(a, j) for a in arrays]
            take = before(ps, arrays) != flip
            arrays = [jnp.where(take, p, a) for p, a in zip(ps, arrays)]
    return arrays


def _nms_kernel(cx1_r, cy1_r, cx2_r, cy2_r, cs_r, clab_r,
                ox1_r, oy1_r, ox2_r, oy2_r, os_r,
                sx1_r, sy1_r, sx2_r, sy2_r, ca_r, acc_r):
    zpad = jnp.zeros((RS - NB, C), jnp.float32)
    x1 = jnp.concatenate([cx1_r[...], zpad], axis=0)
    y1 = jnp.concatenate([cy1_r[...], zpad], axis=0)
    x2 = jnp.concatenate([cx2_r[...], zpad], axis=0)
    y2 = jnp.concatenate([cy2_r[...], zpad], axis=0)
    sc = jnp.concatenate([cs_r[...], zpad - 1.0], axis=0)
    labi = clab_r[...].astype(jnp.int32)
    lab = jnp.concatenate([labi, jnp.zeros((RS - NB, C), jnp.int32)],
                          axis=0)
    lab = jnp.where(_flat_iota() < N, lab, 4)

    maxc = jnp.maximum(jnp.max(cx1_r[...]), jnp.max(cy1_r[...]))
    maxc = jnp.maximum(maxc, jnp.max(cx2_r[...]))
    maxc = jnp.maximum(maxc, jnp.max(cy2_r[...]))
    mc1 = maxc + 1.0

    n0 = jnp.sum((lab == 0).astype(jnp.int32))
    n1 = jnp.sum((labi == 1).astype(jnp.int32))
    n2 = jnp.sum((labi == 2).astype(jnp.int32))
    s1 = n0
    s2 = n0 + n1
    s3 = s2 + n2

    labidx = lab * 65536 + _flat_iota()

    def before1(ps, xs):
        sP, lP = ps[0], ps[1]
        sX, lX = xs[0], xs[1]
        labP = lP >> 16
        labX = lX >> 16
        return (labP < labX) | ((labP == labX)
                                & ((sP > sX) | ((sP == sX) & (lP < lX))))

    ssc, slabidx, x1, y1, x2, y2 = _bitonic([sc, labidx, x1, y1, x2, y2],
                                            before1)
    oidx = slabidx & 65535

    offs = (slabidx >> 16).astype(jnp.float32) * mc1
    sx1 = x1 + offs
    sy1 = y1 + offs
    sx2 = x2 + offs
    sy2 = y2 + offs
    sx1_r[...] = sx1
    sy1_r[...] = sy1
    sx2_r[...] = sx2
    sy2_r[...] = sy2
    ca_r[...] = (sx2 - sx1) * (sy2 - sy1)
    acc_r[...] = jnp.zeros((RS, C), jnp.float32)

    eye = (jax.lax.broadcasted_iota(jnp.int32, (C, C), 0)
           == jax.lax.broadcasted_iota(jnp.int32, (C, C), 1)
           ).astype(jnp.float32)
    tie_diag = (jax.lax.broadcasted_iota(jnp.int32, (C, C), 0)
                < jax.lax.broadcasted_iota(jnp.int32, (C, C), 1))

    def tile_max(ic, rx1, ry1, rx2, ry2, ra):
        c_x1 = sx1_r[pl.ds(ic, 1), :]
        c_y1 = sy1_r[pl.ds(ic, 1), :]
        c_x2 = sx2_r[pl.ds(ic, 1), :]
        c_y2 = sy2_r[pl.ds(ic, 1), :]
        cac = ca_r[pl.ds(ic, 1), :]
        iw = jnp.maximum(jnp.minimum(rx2, c_x2) - jnp.maximum(rx1, c_x1),
                         0.0)
        ih = jnp.maximum(jnp.minimum(ry2, c_y2) - jnp.maximum(ry1, c_y1),
                         0.0)
        inter = iw * ih
        union = jnp.maximum((ra + cac) - inter, 1e-9)
        return inter / union

    def pair(ic, rx1, ry1, rx2, ry2, ra, diag):
        iou = tile_max(ic, rx1, ry1, rx2, ry2, ra)
        if diag:
            iou = jnp.where(tie_diag, iou, 0.0)
        colmax = jnp.max(iou, axis=0, keepdims=True)
        acc_r[pl.ds(ic, 1), :] = jnp.maximum(acc_r[pl.ds(ic, 1), :], colmax)
        return 0

    def outer(ir, _):
        p = ir * C + (C - 1)
        cl = ((p >= s1).astype(jnp.int32) + (p >= s2).astype(jnp.int32)
              + (p >= s3).astype(jnp.int32))
        e = jnp.where(cl == 0, s1,
                      jnp.where(cl == 1, s2,
                                jnp.where(cl == 2, s3, NP)))
        eb = jnp.minimum((e + C - 1) // C, NB)

        q = jnp.concatenate([sx1_r[pl.ds(ir, 1), :], sy1_r[pl.ds(ir, 1), :],
                             sx2_r[pl.ds(ir, 1), :], sy2_r[pl.ds(ir, 1), :],
                             ca_r[pl.ds(ir, 1), :]], axis=0)
        qt = jax.lax.dot_general(eye, q, (((1,), (1,)), ((), ())),
                                 preferred_element_type=jnp.float32,
                                 precision=jax.lax.Precision.HIGHEST)
        rx1 = jnp.broadcast_to(qt[:, 0:1], (C, C))
        ry1 = jnp.broadcast_to(qt[:, 1:2], (C, C))
        rx2 = jnp.broadcast_to(qt[:, 2:3], (C, C))
        ry2 = jnp.broadcast_to(qt[:, 3:4], (C, C))
        ra = jnp.broadcast_to(qt[:, 4:5], (C, C))
        pair(ir, rx1, ry1, rx2, ry2, ra, True)

        def two(u, _):
            t = ir + 1 + 2 * u
            iou1 = tile_max(t, rx1, ry1, rx2, ry2, ra)
            cm1 = jnp.max(iou1, axis=0, keepdims=True)
            acc_r[pl.ds(t, 1), :] = jnp.maximum(acc_r[pl.ds(t, 1), :], cm1)
            t2 = jnp.minimum(t + 1, NB - 1)
            iou2 = tile_max(t2, rx1, ry1, rx2, ry2, ra)
            cm2 = jnp.max(iou2, axis=0, keepdims=True)
            cm2 = jnp.where(t + 1 < eb, cm2, 0.0)
            acc_r[pl.ds(t2, 1), :] = jnp.maximum(acc_r[pl.ds(t2, 1), :],
                                                 cm2)
            return 0

        jax.lax.fori_loop(0, (eb - ir) // 2, two, 0)
        return 0

    jax.lax.fori_loop(0, NB, outer, 0)

    m = acc_r[...]
    s_dec = ssc * jnp.exp(-(m * m) / SIGMA)
    fin = jnp.where(s_dec > SCORE_THRESH, s_dec, 0.0)

    def before2(ps, xs):
        return (ps[0] > xs[0]) | ((ps[0] == xs[0]) & (ps[1] < xs[1]))

    fsrt = _bitonic([fin, oidx, x1, y1, x2, y2], before2)
    ox1_r[...] = fsrt[2][0:3, :]
    oy1_r[...] = fsrt[3][0:3, :]
    ox2_r[...] = fsrt[4][0:3, :]
    oy2_r[...] = fsrt[5][0:3, :]
    os_r[...] = fsrt[0][0:3, :]


def kernel(boxes, scores, labels):
    boxes = boxes.astype(jnp.float32)
    scores = scores.astype(jnp.float32)
    labf = labels.astype(jnp.float32)

    pb = jnp.pad(boxes, ((0, NP - N), (0, 0)))
    ps = jnp.pad(scores, (0, NP - N), constant_values=-1.0)
    plab = jnp.pad(labf, (0, NP - N))

    cols = [pb[:, 0].reshape(NB, C), pb[:, 1].reshape(NB, C),
            pb[:, 2].reshape(NB, C), pb[:, 3].reshape(NB, C),
            ps.reshape(NB, C), plab.reshape(NB, C)]

    outs = pl.pallas_call(
        _nms_kernel,
        out_shape=[jax.ShapeDtypeStruct((3, C), jnp.float32)] * 5,
        scratch_shapes=[pltpu.VMEM((RS, C), jnp.float32)] * 6,
    )(*cols)
    ox1, oy1, ox2, oy2, osc = outs
    topb = jnp.stack([ox1.reshape(-1)[:K], oy1.reshape(-1)[:K],
                      ox2.reshape(-1)[:K], oy2.reshape(-1)[:K]], axis=1)
    return jnp.concatenate([topb, osc.reshape(-1)[:K, None]], axis=1)
